# Initial kernel scaffold; baseline (speedup 1.0000x reference)
import jax
import jax.numpy as jnp
from jax.experimental import pallas as pl


def kernel(node_feature, edge_index, edge_type, node_type, assignment, qs, graph_ids, Ww_rel, Ww_self, bw_gn, Wv_rel, Wv_self, bv_gn, Ww1, bw1, Ww2, bw2, Wv1, bv1, Wv2, bv2):
    raise NotImplementedError("write your pallas kernel here")


# speedup vs baseline: 16.3790x; 16.3790x over previous
import functools

import jax
import jax.numpy as jnp
from jax import lax
from jax.experimental import pallas as pl
from jax.experimental.pallas import tpu as pltpu
from jax.experimental.pallas import tpu_sc as plsc

N = 10000
E = 320000
D = 128
R = 3
G = 32
NSC = 2
NT = 16
SEG = 128
EP = 327680
ET = EP // NT
NCH = ET // SEG
ACC_ROWS = 10240
ACC_PT = ACC_ROWS // NT


EH = EP // NSC
ETD = EH // NT
NCHD = ETD // SEG


def _sc_edge_phase(table, gidx_all, dstp, zacc, onesrc):
    mesh = plsc.VectorSubcoreMesh(core_axis_name="c", subcore_axis_name="s")

    @functools.partial(
        pl.kernel,
        out_type=(
            jax.ShapeDtypeStruct((NSC * ACC_ROWS, D), jnp.float32),
            jax.ShapeDtypeStruct((NSC * ACC_ROWS, D), jnp.float32),
        ),
        mesh=mesh,
        scratch_types=[
            pltpu.VMEM_SHARED((ACC_ROWS, D), jnp.float32),
            pltpu.VMEM((SEG, D), jnp.float32),
            pltpu.VMEM((SEG,), jnp.int32),
            pltpu.VMEM((SEG,), jnp.int32),
            pltpu.VMEM((SEG, D), jnp.float32),
            pltpu.SemaphoreType.DMA,
        ],
    )
    def k(tab_ref, gix_ref, dix_ref, zacc_ref, ones_ref,
          agg_out, deg_out, acc, rows, gidx, dsti, ones, sem):
        cid = lax.axis_index("c")
        sid = lax.axis_index("s")
        my_rows = pl.ds(sid * ACC_PT, ACC_PT)
        o0 = cid * ACC_ROWS + sid * ACC_PT

        pltpu.sync_copy(zacc_ref, acc.at[my_rows])
        pltpu.sync_copy(ones_ref, ones)
        plsc.subcore_barrier()

        ebase = sid * ET

        def chunk_body(kk, carry):
            e0 = ebase + kk * SEG
            pltpu.sync_copy(gix_ref.at[pl.ds(cid * EP + e0, SEG)], gidx)
            pltpu.sync_copy(dix_ref.at[pl.ds(e0, SEG)], dsti)
            pltpu.async_copy(tab_ref.at[gidx], rows, sem).wait()
            pltpu.sync_copy(rows, acc.at[dsti], add=True)
            return carry

        lax.fori_loop(0, NCH, chunk_body, 0)
        plsc.subcore_barrier()

        pltpu.sync_copy(acc.at[my_rows], agg_out.at[pl.ds(o0, ACC_PT)])
        pltpu.sync_copy(zacc_ref, acc.at[my_rows])
        plsc.subcore_barrier()

        dbase = cid * EH + sid * ETD

        def deg_body(kk, carry):
            e0 = dbase + kk * SEG
            pltpu.sync_copy(dix_ref.at[pl.ds(e0, SEG)], dsti)
            pltpu.sync_copy(ones, acc.at[dsti], add=True)
            return carry

        lax.fori_loop(0, NCHD, deg_body, 0)
        plsc.subcore_barrier()

        pltpu.sync_copy(acc.at[my_rows], deg_out.at[pl.ds(o0, ACC_PT)])

    return k(table, gidx_all, dstp, zacc, onesrc)


def _tc_pre_phase(x, Wwr, Wvr):
    B = 1000

    def body(x_ref, wwr_ref, wvr_ref, out_ref):
        xb = x_ref[...]
        wwr = wwr_ref[...]
        wvr = wvr_ref[...]
        for r in range(R):
            out_ref[0, r] = jnp.dot(xb, wwr[r], preferred_element_type=jnp.float32)
            out_ref[1, r] = jnp.dot(xb, wvr[r], preferred_element_type=jnp.float32)

    return pl.pallas_call(
        body,
        grid=(N // B,),
        in_specs=[
            pl.BlockSpec((B, D), lambda i: (i, 0)),
            pl.BlockSpec((R, D, D), lambda i: (0, 0, 0)),
            pl.BlockSpec((R, D, D), lambda i: (0, 0, 0)),
        ],
        out_specs=pl.BlockSpec((NSC, R, B, D), lambda i: (0, 0, i, 0)),
        out_shape=jax.ShapeDtypeStruct((NSC, R, N, D), jnp.float32),
    )(x, Wwr, Wvr)


def _tc_dense_phase(aggw, aggv, x, scal, Wws, Wvs, Ww1, Wv1, w2c, bias):
    B = 1000

    def body(aw_ref, av_ref, x_ref, sc_ref, wws_ref, wvs_ref,
             ww1_ref, wv1_ref, w2_ref, bs_ref, out_ref):
        i = pl.program_id(0)
        xb = x_ref[...]
        sc = sc_ref[...]
        bs = bs_ref[...]

        rdeg = 1.0 / jnp.maximum(sc[:, 0:1], 1.0)
        hw = jnp.maximum(
            aw_ref[...] * rdeg
            + jnp.dot(xb, wws_ref[...], preferred_element_type=jnp.float32)
            + bs[0:1, :], 0.0)
        hv = jnp.maximum(
            av_ref[...] * rdeg
            + jnp.dot(xb, wvs_ref[...], preferred_element_type=jnp.float32)
            + bs[1:2, :], 0.0)

        gw = jnp.maximum(
            jnp.dot(hw, ww1_ref[...], preferred_element_type=jnp.float32)
            + bs[2:3, :], 0.0)
        gv = jnp.maximum(
            jnp.dot(hv, wv1_ref[...], preferred_element_type=jnp.float32)
            + bs[3:4, :], 0.0)

        w2 = w2_ref[...]
        fw = jnp.dot(gw, w2[:, 0:1], preferred_element_type=jnp.float32) + bs[4:5, 0:1]
        fv = jnp.dot(gv, w2[:, 1:2], preferred_element_type=jnp.float32) + bs[4:5, 1:2]

        m = (sc[:, 2:3] == 0.0) & (sc[:, 3:4] == 1.0)
        tot = jnp.where(m, jnp.abs(fw) * sc[:, 1:2] + fv, 0.0)

        gid = sc[:, 4:5]
        li = lax.broadcasted_iota(jnp.int32, (B, D), 1).astype(jnp.float32)
        og = jnp.where(gid == li, 1.0, 0.0)
        contrib = jnp.sum(og * tot, axis=0, keepdims=True)

        @pl.when(i == 0)
        def _():
            out_ref[...] = jnp.zeros((8, D), jnp.float32)

        out_ref[0:1, :] = out_ref[0:1, :] + contrib

    return pl.pallas_call(
        body,
        grid=(N // B,),
        in_specs=[
            pl.BlockSpec((B, D), lambda i: (i, 0)),
            pl.BlockSpec((B, D), lambda i: (i, 0)),
            pl.BlockSpec((B, D), lambda i: (i, 0)),
            pl.BlockSpec((B, 8), lambda i: (i, 0)),
            pl.BlockSpec((D, D), lambda i: (0, 0)),
            pl.BlockSpec((D, D), lambda i: (0, 0)),
            pl.BlockSpec((D, D), lambda i: (0, 0)),
            pl.BlockSpec((D, D), lambda i: (0, 0)),
            pl.BlockSpec((D, 2), lambda i: (0, 0)),
            pl.BlockSpec((8, D), lambda i: (0, 0)),
        ],
        out_specs=pl.BlockSpec((8, D), lambda i: (0, 0)),
        out_shape=jax.ShapeDtypeStruct((8, D), jnp.float32),
    )(aggw, aggv, x, scal, Wws, Wvs, Ww1, Wv1, w2c, bias)


def kernel(node_feature, edge_index, edge_type, node_type, assignment, qs,
           graph_ids, Ww_rel, Ww_self, bw_gn, Wv_rel, Wv_self, bv_gn,
           Ww1, bw1, Ww2, bw2, Wv1, bv1, Wv2, bv2):
    x = node_feature

    table = _tc_pre_phase(x, Ww_rel, Wv_rel).reshape(NSC * R * N, D)

    pad = EP - E
    ar = jnp.arange(pad, dtype=jnp.int32)
    srcp = jnp.concatenate([edge_index[0], ar % 4096])
    dstp = jnp.concatenate([edge_index[1], N + 16 + (ar & 127)])
    typep = jnp.concatenate([edge_type, jnp.zeros((pad,), jnp.int32)])
    gw = srcp + typep * N
    gidx_all = jnp.concatenate([gw, gw + R * N])

    zacc = jnp.zeros((ACC_PT, D), jnp.float32)
    onesrc = jnp.ones((SEG, D), jnp.float32)

    agg_out, deg_out = _sc_edge_phase(table, gidx_all, dstp, zacc, onesrc)

    aggw = agg_out[:N]
    aggv = agg_out[ACC_ROWS:ACC_ROWS + N]
    deg = (deg_out[:N, 0] + deg_out[ACC_ROWS:ACC_ROWS + N, 0])

    z = jnp.zeros((N,), jnp.float32)
    scal = jnp.stack(
        [deg, qs, node_type.astype(jnp.float32), assignment.astype(jnp.float32),
         graph_ids.astype(jnp.float32), z, z, z], axis=1)

    w2c = jnp.concatenate([Ww2, Wv2], axis=1)
    bias = jnp.zeros((8, D), jnp.float32)
    bias = bias.at[0].set(bw_gn).at[1].set(bv_gn).at[2].set(bw1).at[3].set(bv1)
    bias = bias.at[4, 0].set(bw2[0]).at[4, 1].set(bv2[0])

    out8 = _tc_dense_phase(aggw, aggv, x, scal, Ww_self, Wv_self,
                           Ww1, Wv1, w2c, bias)
    return out8[0, :G]


# speedup vs baseline: 24.6543x; 1.5052x over previous
import functools

import jax
import jax.numpy as jnp
from jax import lax
from jax.experimental import pallas as pl
from jax.experimental.pallas import tpu as pltpu
from jax.experimental.pallas import tpu_sc as plsc

N = 10000
E = 320000
D = 128
R = 3
G = 32
NSC = 2
NT = 16
SEG = 128
EP = 327680
ET = EP // NT
NCH = ET // SEG
ACC_ROWS = 10240
ACC_PT = ACC_ROWS // NT


EH = EP // NSC
ETD = EH // NT
NCHD = ETD // SEG


def _sc_edge_phase(table, gidx_all, dstp, zacc, onesrc):
    mesh = plsc.VectorSubcoreMesh(core_axis_name="c", subcore_axis_name="s")
    NB = 2

    @functools.partial(
        pl.kernel,
        out_type=(
            jax.ShapeDtypeStruct((NSC * ACC_ROWS, D), jnp.float32),
            jax.ShapeDtypeStruct((NSC * ACC_ROWS, D), jnp.float32),
        ),
        mesh=mesh,
        scratch_types=[
            pltpu.VMEM_SHARED((ACC_ROWS, D), jnp.float32),
            pltpu.VMEM((NB, SEG, D), jnp.float32),
            [pltpu.VMEM((SEG,), jnp.int32) for _ in range(NB)],
            [pltpu.VMEM((SEG,), jnp.int32) for _ in range(NB)],
            [pltpu.SemaphoreType.DMA for _ in range(NB)],
            [pltpu.SemaphoreType.DMA for _ in range(NB)],
        ],
    )
    def k(tab_ref, gix_ref, dix_ref, zacc_ref, ones_ref,
          agg_out, deg_out, acc, rows, gidx, dsti, gsem, ssem):
        cid = lax.axis_index("c")
        sid = lax.axis_index("s")
        my_rows = pl.ds(sid * ACC_PT, ACC_PT)
        o0 = cid * ACC_ROWS + sid * ACC_PT

        pltpu.sync_copy(zacc_ref, acc.at[my_rows])
        plsc.subcore_barrier()

        ebase = sid * ET

        def stage_and_gather(b, c):
            e0 = ebase + c * SEG
            pltpu.sync_copy(gix_ref.at[pl.ds(cid * EP + e0, SEG)], gidx[b])
            pltpu.sync_copy(dix_ref.at[pl.ds(e0, SEG)], dsti[b])
            pltpu.async_copy(tab_ref.at[gidx[b]], rows.at[b], gsem[b])

        for b in range(NB):
            stage_and_gather(b, b)

        def chunk_body(g, carry):
            scs = []
            for b in range(NB):
                pltpu.make_async_copy(tab_ref.at[gidx[b]], rows.at[b],
                                      gsem[b]).wait()
                scs.append(pltpu.async_copy(rows.at[b], acc.at[dsti[b]],
                                            ssem[b], add=True))
            for b in range(NB):
                scs[b].wait()
                nxt = g * NB + b + NB

                @pl.when(nxt < NCH)
                def _():
                    stage_and_gather(b, nxt)

            return carry

        lax.fori_loop(0, NCH // NB, chunk_body, 0)
        plsc.subcore_barrier()

        pltpu.sync_copy(acc.at[my_rows], agg_out.at[pl.ds(o0, ACC_PT)])
        pltpu.sync_copy(zacc_ref, acc.at[my_rows])
        ones = rows.at[0]
        pltpu.sync_copy(ones_ref, ones)
        plsc.subcore_barrier()

        dbase = cid * EH + sid * ETD

        def stage_dst(b, c):
            pltpu.sync_copy(dix_ref.at[pl.ds(dbase + c * SEG, SEG)], dsti[b])

        stage_dst(0, 0)

        def deg_body(gg, carry):
            pltpu.async_copy(ones, acc.at[dsti[0]], ssem[0], add=True)

            @pl.when(gg > 0)
            def _():
                pltpu.make_async_copy(ones, acc.at[dsti[1]], ssem[1]).wait()

            stage_dst(1, 2 * gg + 1)
            pltpu.async_copy(ones, acc.at[dsti[1]], ssem[1], add=True)
            pltpu.make_async_copy(ones, acc.at[dsti[0]], ssem[0]).wait()

            @pl.when(gg + 1 < NCHD // 2)
            def _():
                stage_dst(0, 2 * gg + 2)

            return carry

        lax.fori_loop(0, NCHD // 2, deg_body, 0)
        pltpu.make_async_copy(ones, acc.at[dsti[1]], ssem[1]).wait()
        plsc.subcore_barrier()

        pltpu.sync_copy(acc.at[my_rows], deg_out.at[pl.ds(o0, ACC_PT)])

    return k(table, gidx_all, dstp, zacc, onesrc)


def _tc_pre_phase(x, Wwr, Wvr):
    B = 1000

    def body(x_ref, wwr_ref, wvr_ref, out_ref):
        xb = x_ref[...]
        wwr = wwr_ref[...]
        wvr = wvr_ref[...]
        for r in range(R):
            out_ref[0, r] = jnp.dot(xb, wwr[r], preferred_element_type=jnp.float32)
            out_ref[1, r] = jnp.dot(xb, wvr[r], preferred_element_type=jnp.float32)

    return pl.pallas_call(
        body,
        grid=(N // B,),
        in_specs=[
            pl.BlockSpec((B, D), lambda i: (i, 0)),
            pl.BlockSpec((R, D, D), lambda i: (0, 0, 0)),
            pl.BlockSpec((R, D, D), lambda i: (0, 0, 0)),
        ],
        out_specs=pl.BlockSpec((NSC, R, B, D), lambda i: (0, 0, i, 0)),
        out_shape=jax.ShapeDtypeStruct((NSC, R, N, D), jnp.float32),
    )(x, Wwr, Wvr)


def _tc_dense_phase(aggw, aggv, x, scal, Wws, Wvs, Ww1, Wv1, w2c, bias):
    B = 1000

    def body(aw_ref, av_ref, x_ref, sc_ref, wws_ref, wvs_ref,
             ww1_ref, wv1_ref, w2_ref, bs_ref, out_ref):
        i = pl.program_id(0)
        xb = x_ref[...]
        sc = sc_ref[...]
        bs = bs_ref[...]

        rdeg = 1.0 / jnp.maximum(sc[:, 0:1], 1.0)
        hw = jnp.maximum(
            aw_ref[...] * rdeg
            + jnp.dot(xb, wws_ref[...], preferred_element_type=jnp.float32)
            + bs[0:1, :], 0.0)
        hv = jnp.maximum(
            av_ref[...] * rdeg
            + jnp.dot(xb, wvs_ref[...], preferred_element_type=jnp.float32)
            + bs[1:2, :], 0.0)

        gw = jnp.maximum(
            jnp.dot(hw, ww1_ref[...], preferred_element_type=jnp.float32)
            + bs[2:3, :], 0.0)
        gv = jnp.maximum(
            jnp.dot(hv, wv1_ref[...], preferred_element_type=jnp.float32)
            + bs[3:4, :], 0.0)

        w2 = w2_ref[...]
        fw = jnp.dot(gw, w2[:, 0:1], preferred_element_type=jnp.float32) + bs[4:5, 0:1]
        fv = jnp.dot(gv, w2[:, 1:2], preferred_element_type=jnp.float32) + bs[4:5, 1:2]

        m = (sc[:, 2:3] == 0.0) & (sc[:, 3:4] == 1.0)
        tot = jnp.where(m, jnp.abs(fw) * sc[:, 1:2] + fv, 0.0)

        gid = sc[:, 4:5]
        li = lax.broadcasted_iota(jnp.int32, (B, D), 1).astype(jnp.float32)
        og = jnp.where(gid == li, 1.0, 0.0)
        contrib = jnp.sum(og * tot, axis=0, keepdims=True)

        @pl.when(i == 0)
        def _():
            out_ref[...] = jnp.zeros((8, D), jnp.float32)

        out_ref[0:1, :] = out_ref[0:1, :] + contrib

    return pl.pallas_call(
        body,
        grid=(N // B,),
        in_specs=[
            pl.BlockSpec((B, D), lambda i: (i, 0)),
            pl.BlockSpec((B, D), lambda i: (i, 0)),
            pl.BlockSpec((B, D), lambda i: (i, 0)),
            pl.BlockSpec((B, 8), lambda i: (i, 0)),
            pl.BlockSpec((D, D), lambda i: (0, 0)),
            pl.BlockSpec((D, D), lambda i: (0, 0)),
            pl.BlockSpec((D, D), lambda i: (0, 0)),
            pl.BlockSpec((D, D), lambda i: (0, 0)),
            pl.BlockSpec((D, 2), lambda i: (0, 0)),
            pl.BlockSpec((8, D), lambda i: (0, 0)),
        ],
        out_specs=pl.BlockSpec((8, D), lambda i: (0, 0)),
        out_shape=jax.ShapeDtypeStruct((8, D), jnp.float32),
    )(aggw, aggv, x, scal, Wws, Wvs, Ww1, Wv1, w2c, bias)


def kernel(node_feature, edge_index, edge_type, node_type, assignment, qs,
           graph_ids, Ww_rel, Ww_self, bw_gn, Wv_rel, Wv_self, bv_gn,
           Ww1, bw1, Ww2, bw2, Wv1, bv1, Wv2, bv2):
    x = node_feature

    table = _tc_pre_phase(x, Ww_rel, Wv_rel).reshape(NSC * R * N, D)

    pad = EP - E
    ar = jnp.arange(pad, dtype=jnp.int32)
    srcp = jnp.concatenate([edge_index[0], ar % 4096])
    dstp = jnp.concatenate([edge_index[1], N + 16 + (ar & 127)])
    typep = jnp.concatenate([edge_type, jnp.zeros((pad,), jnp.int32)])
    gw = srcp + typep * N
    gidx_all = jnp.concatenate([gw, gw + R * N])

    zacc = jnp.zeros((ACC_PT, D), jnp.float32)
    onesrc = jnp.ones((SEG, D), jnp.float32)

    agg_out, deg_out = _sc_edge_phase(table, gidx_all, dstp, zacc, onesrc)

    aggw = agg_out[:N]
    aggv = agg_out[ACC_ROWS:ACC_ROWS + N]
    deg = (deg_out[:N, 0] + deg_out[ACC_ROWS:ACC_ROWS + N, 0])

    z = jnp.zeros((N,), jnp.float32)
    scal = jnp.stack(
        [deg, qs, node_type.astype(jnp.float32), assignment.astype(jnp.float32),
         graph_ids.astype(jnp.float32), z, z, z], axis=1)

    w2c = jnp.concatenate([Ww2, Wv2], axis=1)
    bias = jnp.zeros((8, D), jnp.float32)
    bias = bias.at[0].set(bw_gn).at[1].set(bv_gn).at[2].set(bw1).at[3].set(bv1)
    bias = bias.at[4, 0].set(bw2[0]).at[4, 1].set(bv2[0])

    out8 = _tc_dense_phase(aggw, aggv, x, scal, Ww_self, Wv_self,
                           Ww1, Wv1, w2c, bias)
    return out8[0, :G]
